# baseline (speedup 1.0000x reference)
import functools

import jax
import jax.numpy as jnp
from jax import lax
from jax.experimental import pallas as pl
from jax.experimental.pallas import tpu as pltpu
from jax.experimental.pallas import tpu_sc as plsc

N = 10000
E = 320000
D = 128
LAYERS = 3

NP = 10240
NSC = 2
NTILE = 16
NW = NSC * NTILE
EPW = E // NW
CH = 80
NCHUNK = EPW // CH
DH = D // NSC
NCHT = E // CH // NTILE
RPT = NP // NTILE

_mesh = plsc.VectorSubcoreMesh(core_axis_name="c", subcore_axis_name="s")


def _deg_body(col2_hbm, degp_hbm, cidx_v, ones_v, zrow_v, deg_sh, sem):
    c = lax.axis_index("c")
    s = lax.axis_index("s")
    wid = c * NTILE + s

    for j in range(CH // 16):
        ones_v[pl.ds(j * 16, 16)] = jnp.ones((16,), jnp.float32)
    for j in range(RPT // 16):
        zrow_v[pl.ds(j * 16, 16)] = jnp.zeros((16,), jnp.float32)
    pltpu.sync_copy(zrow_v, deg_sh.at[pl.ds(s * RPT, RPT)])
    pltpu.sync_copy(col2_hbm.at[wid], cidx_v)
    plsc.subcore_barrier()

    K = 25

    def batch(b, carry):
        def fire(i, carry2):
            pltpu.async_copy(ones_v, deg_sh.at[cidx_v.at[b * K + i]], sem,
                             add=True)
            return carry2

        lax.fori_loop(0, K, fire, 0)

        def drain(i, carry2):
            pltpu.make_async_copy(
                ones_v, deg_sh.at[cidx_v.at[0]], sem).wait()
            return carry2

        lax.fori_loop(0, K, drain, 0)
        return carry

    lax.fori_loop(0, NCHUNK // K, batch, 0)
    plsc.subcore_barrier()
    pltpu.sync_copy(deg_sh.at[pl.ds(s * RPT, RPT)],
                    degp_hbm.at[c, pl.ds(s * RPT, RPT)])


_deg_kernel = pl.kernel(
    _deg_body,
    out_type=jax.ShapeDtypeStruct((NSC, NP), jnp.float32),
    mesh=_mesh,
    scratch_types=[
        pltpu.VMEM((NCHUNK, CH), jnp.int32),
        pltpu.VMEM((CH,), jnp.float32),
        pltpu.VMEM((RPT,), jnp.float32),
        pltpu.VMEM_SHARED((NP,), jnp.float32),
        pltpu.SemaphoreType.DMA,
    ],
)


NB = 5
GK = 3


def _sweep_body(zst_hbm, row16_hbm, col16_hbm, part_hbm,
                ridx, cidx, *bufs):
    rows = list(bufs[:NB])
    zbuf = bufs[NB]
    acc_sh = bufs[NB + 1]
    gsem = list(bufs[NB + 2:2 * NB + 2])
    ssem = list(bufs[2 * NB + 2:])
    c = lax.axis_index("c")
    s = lax.axis_index("s")

    zsrc = zst_hbm.at[c]

    pltpu.sync_copy(row16_hbm.at[s], ridx)
    pltpu.sync_copy(col16_hbm.at[s], cidx)

    ZB = 64

    def zb(i, carry):
        for j in range(DH // 16):
            zbuf[i, pl.ds(j * 16, 16)] = jnp.zeros((16,), jnp.float32)
        return carry

    lax.fori_loop(0, ZB, zb, 0)

    def zc(i, carry):
        pltpu.sync_copy(zbuf, acc_sh.at[pl.ds(s * RPT + i * ZB, ZB)])
        return carry

    lax.fori_loop(0, RPT // ZB, zc, 0)
    plsc.subcore_barrier()

    def gather(i, b):
        pltpu.async_copy(zsrc.at[ridx.at[i]], rows[b], gsem[b])

    def gwait(b):
        pltpu.make_async_copy(zsrc.at[ridx.at[0]], rows[b], gsem[b]).wait()

    def scat(i, b):
        pltpu.async_copy(rows[b], acc_sh.at[cidx.at[i]], ssem[b], add=True)

    def swait(b):
        pltpu.make_async_copy(rows[b], acc_sh.at[cidx.at[0]], ssem[b]).wait()

    for b in range(GK):
        gather(b, b)

    def step(go, first, last):
        g = go * NB
        for b in range(NB):
            i = g + b
            gwait(b)
            scat(i, b)
            bk = (b + GK) % NB
            if last and b >= NB - GK:
                continue
            if not (first and b < NB - GK):
                swait(bk)
            gather(i + GK, bk)

    step(0, True, False)

    def mid(go, carry):
        step(go, False, False)
        return carry

    lax.fori_loop(1, NCHT // NB - 1, mid, 0)
    step(NCHT // NB - 1, False, True)

    for b in range(NB):
        swait(b)
    plsc.subcore_barrier()

    def wo(i, carry):
        rr = s * RPT + i * ZB
        pltpu.sync_copy(acc_sh.at[pl.ds(rr, ZB)],
                        part_hbm.at[c, pl.ds(rr, ZB)])
        return carry

    lax.fori_loop(0, RPT // ZB, wo, 0)


_sweep_kernel = pl.kernel(
    _sweep_body,
    out_type=jax.ShapeDtypeStruct((NSC, NP, DH), jnp.float32),
    mesh=_mesh,
    scratch_types=(
        [pltpu.VMEM((NCHT, CH), jnp.int32)] * 2
        + [pltpu.VMEM((CH, DH), jnp.float32)] * NB
        + [pltpu.VMEM((64, DH), jnp.float32),
           pltpu.VMEM_SHARED((NP, DH), jnp.float32)]
        + [pltpu.SemaphoreType.DMA] * (2 * NB)
    ),
    compiler_params=pltpu.CompilerParams(use_tc_tiling_on_sc=False),
)


_BLK = 512
_GRID = NP // _BLK


def _scale_body(degp_ref, emb_ref, dis_ref, z0_ref):
    deg = degp_ref[0] + degp_ref[1]
    dis = jnp.where(deg > 0, lax.rsqrt(jnp.maximum(deg, 1.0)), 0.0)
    dis_ref[...] = dis
    z0_ref[0] = dis * emb_ref[:, :DH]
    z0_ref[1] = dis * emb_ref[:, DH:]


_scale_kernel = pl.pallas_call(
    _scale_body,
    grid=(_GRID,),
    in_specs=[
        pl.BlockSpec((NSC, _BLK, 1), lambda i: (0, i, 0)),
        pl.BlockSpec((_BLK, D), lambda i: (i, 0)),
    ],
    out_specs=[
        pl.BlockSpec((_BLK, 1), lambda i: (i, 0)),
        pl.BlockSpec((NSC, _BLK, DH), lambda i: (0, i, 0)),
    ],
    out_shape=[
        jax.ShapeDtypeStruct((NP, 1), jnp.float32),
        jax.ShapeDtypeStruct((NSC, NP, DH), jnp.float32),
    ],
)


def _combine_body(part_ref, dis_ref, osum_ref, z_ref, osum_out_ref):
    dis = dis_ref[...]
    da0 = dis * part_ref[0]
    da1 = dis * part_ref[1]
    z_ref[0] = dis * da0
    z_ref[1] = dis * da1
    osum_out_ref[:, :DH] = osum_ref[:, :DH] + da0 * (1.0 / LAYERS)
    osum_out_ref[:, DH:] = osum_ref[:, DH:] + da1 * (1.0 / LAYERS)


_combine_kernel = pl.pallas_call(
    _combine_body,
    grid=(_GRID,),
    in_specs=[
        pl.BlockSpec((NSC, _BLK, DH), lambda i: (0, i, 0)),
        pl.BlockSpec((_BLK, 1), lambda i: (i, 0)),
        pl.BlockSpec((_BLK, D), lambda i: (i, 0)),
    ],
    out_specs=[
        pl.BlockSpec((NSC, _BLK, DH), lambda i: (0, i, 0)),
        pl.BlockSpec((_BLK, D), lambda i: (i, 0)),
    ],
    out_shape=[
        jax.ShapeDtypeStruct((NSC, NP, DH), jnp.float32),
        jax.ShapeDtypeStruct((NP, D), jnp.float32),
    ],
)


def kernel(x2, edge_index, emb_weight):
    del x2
    col32 = edge_index[1].astype(jnp.int32).reshape(NW, NCHUNK, CH)
    row16 = edge_index[0].astype(jnp.int32).reshape(NTILE, NCHT, CH)
    col16 = edge_index[1].astype(jnp.int32).reshape(NTILE, NCHT, CH)
    emb_pad = jnp.zeros((NP, D), jnp.float32).at[:N].set(emb_weight)

    degp = _deg_kernel(col32)
    degp = degp.reshape(NSC, NP, 1)
    dis, z = _scale_kernel(degp, emb_pad)

    osum = jnp.zeros((NP, D), jnp.float32)
    for _ in range(LAYERS):
        part = _sweep_kernel(z, row16, col16)
        z, osum = _combine_kernel(part, dis, osum)

    return osum[:N]


# speedup vs baseline: 1.1987x; 1.1987x over previous
import functools

import jax
import jax.numpy as jnp
from jax import lax
from jax.experimental import pallas as pl
from jax.experimental.pallas import tpu as pltpu
from jax.experimental.pallas import tpu_sc as plsc

N = 10000
E = 320000
D = 128
LAYERS = 3

NP = 10240
NSC = 2
NTILE = 16
NW = NSC * NTILE
EPW = E // NW
CH = 80
NCHUNK = EPW // CH
DH = D // NSC
NCHT = E // CH // NTILE
RPT = NP // NTILE

_mesh = plsc.VectorSubcoreMesh(core_axis_name="c", subcore_axis_name="s")


def _deg_body(col2_hbm, degp_hbm, cidx_v, ones_v, zrow_v, deg_sh, sem):
    c = lax.axis_index("c")
    s = lax.axis_index("s")
    wid = c * NTILE + s

    for j in range(CH // 16):
        ones_v[pl.ds(j * 16, 16)] = jnp.ones((16,), jnp.float32)
    for j in range(RPT // 16):
        zrow_v[pl.ds(j * 16, 16)] = jnp.zeros((16,), jnp.float32)
    pltpu.sync_copy(zrow_v, deg_sh.at[pl.ds(s * RPT, RPT)])
    pltpu.sync_copy(col2_hbm.at[wid], cidx_v)
    plsc.subcore_barrier()

    K = 25

    def batch(b, carry):
        def fire(i, carry2):
            pltpu.async_copy(ones_v, deg_sh.at[cidx_v.at[b * K + i]], sem,
                             add=True)
            return carry2

        lax.fori_loop(0, K, fire, 0)

        def drain(i, carry2):
            pltpu.make_async_copy(
                ones_v, deg_sh.at[cidx_v.at[0]], sem).wait()
            return carry2

        lax.fori_loop(0, K, drain, 0)
        return carry

    lax.fori_loop(0, NCHUNK // K, batch, 0)
    plsc.subcore_barrier()
    pltpu.sync_copy(deg_sh.at[pl.ds(s * RPT, RPT)],
                    degp_hbm.at[c, pl.ds(s * RPT, RPT)])


_deg_kernel = pl.kernel(
    _deg_body,
    out_type=jax.ShapeDtypeStruct((NSC, NP), jnp.float32),
    mesh=_mesh,
    scratch_types=[
        pltpu.VMEM((NCHUNK, CH), jnp.int32),
        pltpu.VMEM((CH,), jnp.float32),
        pltpu.VMEM((RPT,), jnp.float32),
        pltpu.VMEM_SHARED((NP,), jnp.float32),
        pltpu.SemaphoreType.DMA,
    ],
)


NB = 5
GK = 4


def _sweep_body(zst_hbm, row16_hbm, col16_hbm, dis_hbm, osum_in_hbm,
                z_out, osum_out, ridx, cidx, *bufs):
    rows = list(bufs[:NB])
    zbuf = bufs[NB]
    obuf = bufs[NB + 1]
    dis_v = bufs[NB + 2]
    acc_sh = bufs[NB + 3]
    gsem = list(bufs[NB + 4:2 * NB + 4])
    ssem = list(bufs[2 * NB + 4:])
    c = lax.axis_index("c")
    s = lax.axis_index("s")

    zsrc = zst_hbm.at[c]

    pltpu.sync_copy(row16_hbm.at[s], ridx)
    pltpu.sync_copy(col16_hbm.at[s], cidx)
    pltpu.sync_copy(dis_hbm.at[pl.ds(s * RPT, RPT)], dis_v)

    ZB = 64

    def zb(i, carry):
        for j in range(DH // 16):
            zbuf[i, pl.ds(j * 16, 16)] = jnp.zeros((16,), jnp.float32)
        return carry

    lax.fori_loop(0, ZB, zb, 0)

    def zc(i, carry):
        pltpu.sync_copy(zbuf, acc_sh.at[pl.ds(s * RPT + i * ZB, ZB)])
        return carry

    lax.fori_loop(0, RPT // ZB, zc, 0)
    plsc.subcore_barrier()

    def gather(i, b):
        pltpu.async_copy(zsrc.at[ridx.at[i]], rows[b], gsem[b])

    def gwait(b):
        pltpu.make_async_copy(zsrc.at[ridx.at[0]], rows[b], gsem[b]).wait()

    def scat(i, b):
        pltpu.async_copy(rows[b], acc_sh.at[cidx.at[i]], ssem[b], add=True)

    def swait(b):
        pltpu.make_async_copy(rows[b], acc_sh.at[cidx.at[0]], ssem[b]).wait()

    for b in range(GK):
        gather(b, b)

    def step(go, first, last):
        g = go * NB
        for b in range(NB):
            i = g + b
            gwait(b)
            scat(i, b)
            bk = (b + GK) % NB
            if last and b >= NB - GK:
                continue
            if not (first and b < NB - GK):
                swait(bk)
            gather(i + GK, bk)

    step(0, True, False)

    def mid(go, carry):
        step(go, False, False)
        return carry

    lax.fori_loop(1, NCHT // NB - 1, mid, 0)
    step(NCHT // NB - 1, False, True)

    for b in range(NB):
        swait(b)
    plsc.subcore_barrier()

    def wo(i, carry):
        rr = s * RPT + i * ZB
        pltpu.sync_copy(acc_sh.at[pl.ds(rr, ZB)], zbuf)
        pltpu.sync_copy(osum_in_hbm.at[c, pl.ds(rr, ZB)], obuf)

        def rowgrp(g, carry2):
            dv = dis_v[pl.ds(i * ZB + g * 16, 16)]
            for k in range(16):
                r = g * 16 + k
                d = dv[k]
                d2 = d * d
                d3 = d * (1.0 / LAYERS)
                for j in range(DH // 16):
                    a = zbuf[r, pl.ds(j * 16, 16)]
                    zbuf[r, pl.ds(j * 16, 16)] = d2 * a
                    o = obuf[r, pl.ds(j * 16, 16)]
                    obuf[r, pl.ds(j * 16, 16)] = o + d3 * a
            return carry2

        lax.fori_loop(0, ZB // 16, rowgrp, 0)
        pltpu.sync_copy(zbuf, z_out.at[c, pl.ds(rr, ZB)])
        pltpu.sync_copy(obuf, osum_out.at[c, pl.ds(rr, ZB)])
        return carry

    lax.fori_loop(0, RPT // ZB, wo, 0)


_sweep_kernel = pl.kernel(
    _sweep_body,
    out_type=[jax.ShapeDtypeStruct((NSC, NP, DH), jnp.float32),
              jax.ShapeDtypeStruct((NSC, NP, DH), jnp.float32)],
    mesh=_mesh,
    scratch_types=(
        [pltpu.VMEM((NCHT, CH), jnp.int32)] * 2
        + [pltpu.VMEM((CH, DH), jnp.float32)] * NB
        + [pltpu.VMEM((64, DH), jnp.float32),
           pltpu.VMEM((64, DH), jnp.float32),
           pltpu.VMEM((RPT,), jnp.float32),
           pltpu.VMEM_SHARED((NP, DH), jnp.float32)]
        + [pltpu.SemaphoreType.DMA] * (2 * NB)
    ),
    compiler_params=pltpu.CompilerParams(use_tc_tiling_on_sc=False),
)


_BLK = 512
_GRID = NP // _BLK


def _scale_body(degp_ref, emb_ref, dis_ref, z0_ref):
    deg = degp_ref[0] + degp_ref[1]
    dis = jnp.where(deg > 0, lax.rsqrt(jnp.maximum(deg, 1.0)), 0.0)
    dis_ref[...] = dis
    z0_ref[0] = dis * emb_ref[:, :DH]
    z0_ref[1] = dis * emb_ref[:, DH:]


_scale_kernel = pl.pallas_call(
    _scale_body,
    grid=(_GRID,),
    in_specs=[
        pl.BlockSpec((NSC, _BLK, 1), lambda i: (0, i, 0)),
        pl.BlockSpec((_BLK, D), lambda i: (i, 0)),
    ],
    out_specs=[
        pl.BlockSpec((_BLK, 1), lambda i: (i, 0)),
        pl.BlockSpec((NSC, _BLK, DH), lambda i: (0, i, 0)),
    ],
    out_shape=[
        jax.ShapeDtypeStruct((NP, 1), jnp.float32),
        jax.ShapeDtypeStruct((NSC, NP, DH), jnp.float32),
    ],
)


def kernel(x2, edge_index, emb_weight):
    del x2
    col32 = edge_index[1].astype(jnp.int32).reshape(NW, NCHUNK, CH)
    row16 = edge_index[0].astype(jnp.int32).reshape(NTILE, NCHT, CH)
    col16 = edge_index[1].astype(jnp.int32).reshape(NTILE, NCHT, CH)
    emb_pad = jnp.zeros((NP, D), jnp.float32).at[:N].set(emb_weight)

    degp = _deg_kernel(col32)
    degp = degp.reshape(NSC, NP, 1)
    dis, z = _scale_kernel(degp, emb_pad)
    dis = dis.reshape(NP)

    osum = jnp.zeros((NSC, NP, DH), jnp.float32)
    for _ in range(LAYERS):
        z, osum = _sweep_kernel(z, row16, col16, dis, osum)

    return jnp.concatenate([osum[0, :N], osum[1, :N]], axis=1)


# speedup vs baseline: 1.2960x; 1.0812x over previous
import jax
import jax.numpy as jnp
from jax import lax
from jax.experimental import pallas as pl
from jax.experimental.pallas import tpu as pltpu
from jax.experimental.pallas import tpu_sc as plsc

N = 10000
E = 320000
D = 128
LAYERS = 3

NP = 10240
NSC = 2
NTILE = 16
CH = 80
NCHT = E // CH // NTILE
DH = D // NSC
RPT = NP // NTILE
ZB = 64
NBLK = RPT // ZB

NB = 5
GK = 4

_mesh = plsc.VectorSubcoreMesh(core_axis_name="c", subcore_axis_name="s")


def _rsqrt16(x):
    i = plsc.bitcast(x, jnp.int32)
    y = plsc.bitcast(jnp.int32(0x5F3759DF) - (i >> 1), jnp.float32)
    for _ in range(3):
        y = y * (1.5 - 0.5 * x * y * y)
    return jnp.where(x > 0.0, y, 0.0)


def _mega_body(emb2_hbm, row16_hbm, col16_hbm,
               osum_hbm, zp_hbm, zq_hbm,
               ridx, cidx, *bufs):
    rows = list(bufs[:NB])
    zbuf = bufs[NB]
    zerob = bufs[NB + 1]
    obuf = bufs[NB + 2]
    ones_v = bufs[NB + 3]
    dis_v = bufs[NB + 4]
    deg_sh = bufs[NB + 5]
    acc_sh = bufs[NB + 6]
    hsem = bufs[NB + 7]
    gsem = list(bufs[NB + 8:2 * NB + 8])
    ssem = list(bufs[2 * NB + 8:])
    c = lax.axis_index("c")
    s = lax.axis_index("s")
    rr0 = s * RPT

    pltpu.sync_copy(row16_hbm.at[s], ridx)
    pltpu.sync_copy(col16_hbm.at[s], cidx)

    for j in range(CH // 16):
        ones_v[pl.ds(j * 16, 16)] = jnp.ones((16,), jnp.float32)

    def zb(i, carry):
        for j in range(DH // 16):
            zerob[i, pl.ds(j * 16, 16)] = jnp.zeros((16,), jnp.float32)
        return carry

    lax.fori_loop(0, ZB, zb, 0)

    def zd(i, carry):
        dis_v[pl.ds(i * 16, 16)] = jnp.zeros((16,), jnp.float32)
        return carry

    lax.fori_loop(0, RPT // 16, zd, 0)
    pltpu.sync_copy(dis_v, deg_sh.at[pl.ds(rr0, RPT)])

    def zc(i, carry):
        pltpu.sync_copy(zerob, acc_sh.at[pl.ds(rr0 + i * ZB, ZB)])
        return carry

    lax.fori_loop(0, NBLK, zc, 0)
    plsc.subcore_barrier()

    HK = 25

    def hbatch(b, carry):
        def fire(i, carry2):
            pltpu.async_copy(ones_v, deg_sh.at[cidx.at[b * HK + i]], hsem,
                             add=True)
            return carry2

        lax.fori_loop(0, HK, fire, 0)

        def drain(i, carry2):
            pltpu.make_async_copy(ones_v, deg_sh.at[cidx.at[0]], hsem).wait()
            return carry2

        lax.fori_loop(0, HK, drain, 0)
        return carry

    lax.fori_loop(0, NCHT // HK, hbatch, 0)
    plsc.subcore_barrier()

    pltpu.sync_copy(deg_sh.at[pl.ds(rr0, RPT)], dis_v)

    def newt(k, carry):
        x = dis_v[pl.ds(k * 16, 16)]
        dis_v[pl.ds(k * 16, 16)] = _rsqrt16(x)
        return carry

    lax.fori_loop(0, RPT // 16, newt, 0)

    def z0blk(i, carry):
        rr = rr0 + i * ZB
        pltpu.sync_copy(emb2_hbm.at[c, pl.ds(rr, ZB)], zbuf)

        def rowgrp(g, carry2):
            dv = dis_v[pl.ds(i * ZB + g * 16, 16)]
            for k in range(16):
                r = g * 16 + k
                d = dv[k]
                for j in range(DH // 16):
                    a = zbuf[r, pl.ds(j * 16, 16)]
                    zbuf[r, pl.ds(j * 16, 16)] = d * a
            return carry2

        lax.fori_loop(0, ZB // 16, rowgrp, 0)
        pltpu.sync_copy(zbuf, zp_hbm.at[c, pl.ds(rr, ZB)])
        return carry

    lax.fori_loop(0, NBLK, z0blk, 0)
    plsc.subcore_barrier()

    def sweep(zsrc):
        def gather(i, b):
            pltpu.async_copy(zsrc.at[ridx.at[i]], rows[b], gsem[b])

        def gwait(b):
            pltpu.make_async_copy(zsrc.at[ridx.at[0]], rows[b],
                                  gsem[b]).wait()

        def scat(i, b):
            pltpu.async_copy(rows[b], acc_sh.at[cidx.at[i]], ssem[b],
                             add=True)

        def swait(b):
            pltpu.make_async_copy(rows[b], acc_sh.at[cidx.at[0]],
                                  ssem[b]).wait()

        for b in range(GK):
            gather(b, b)

        def step(go, first, last):
            g = go * NB
            for b in range(NB):
                i = g + b
                gwait(b)
                scat(i, b)
                bk = (b + GK) % NB
                if last and b >= NB - GK:
                    continue
                if not (first and b < NB - GK):
                    swait(bk)
                gather(i + GK, bk)

        step(0, True, False)

        def mid(go, carry):
            step(go, False, False)
            return carry

        lax.fori_loop(1, NCHT // NB - 1, mid, 0)
        step(NCHT // NB - 1, False, True)
        for b in range(NB):
            swait(b)

    for layer in range(LAYERS):
        zin, zot = (zp_hbm, zq_hbm) if layer % 2 == 0 else (zq_hbm, zp_hbm)
        sweep(zin.at[c])
        plsc.subcore_barrier()

        last_layer = layer == LAYERS - 1

        first_layer = layer == 0

        def wo(i, carry):
            rr = rr0 + i * ZB
            pltpu.sync_copy(acc_sh.at[pl.ds(rr, ZB)], zbuf)
            if not first_layer:
                pltpu.sync_copy(osum_hbm.at[c, pl.ds(rr, ZB)], obuf)

            def rowgrp(g, carry2):
                dv = dis_v[pl.ds(i * ZB + g * 16, 16)]
                for k in range(16):
                    r = g * 16 + k
                    d = dv[k]
                    d2 = d * d
                    d3 = d * (1.0 / LAYERS)
                    for j in range(DH // 16):
                        a = zbuf[r, pl.ds(j * 16, 16)]
                        if first_layer:
                            obuf[r, pl.ds(j * 16, 16)] = d3 * a
                        else:
                            o = obuf[r, pl.ds(j * 16, 16)]
                            obuf[r, pl.ds(j * 16, 16)] = o + d3 * a
                        if not last_layer:
                            zbuf[r, pl.ds(j * 16, 16)] = d2 * a
                return carry2

            lax.fori_loop(0, ZB // 16, rowgrp, 0)
            pltpu.sync_copy(obuf, osum_hbm.at[c, pl.ds(rr, ZB)])
            if not last_layer:
                pltpu.sync_copy(zbuf, zot.at[c, pl.ds(rr, ZB)])
                pltpu.sync_copy(zerob, acc_sh.at[pl.ds(rr, ZB)])
            return carry

        lax.fori_loop(0, NBLK, wo, 0)
        plsc.subcore_barrier()


_mega_kernel = pl.kernel(
    _mega_body,
    out_type=[jax.ShapeDtypeStruct((NSC, NP, DH), jnp.float32),
              jax.ShapeDtypeStruct((NSC, NP, DH), jnp.float32),
              jax.ShapeDtypeStruct((NSC, NP, DH), jnp.float32)],
    mesh=_mesh,
    scratch_types=(
        [pltpu.VMEM((NCHT, CH), jnp.int32)] * 2
        + [pltpu.VMEM((CH, DH), jnp.float32)] * NB
        + [pltpu.VMEM((ZB, DH), jnp.float32),
           pltpu.VMEM((ZB, DH), jnp.float32),
           pltpu.VMEM((ZB, DH), jnp.float32),
           pltpu.VMEM((CH,), jnp.float32),
           pltpu.VMEM((RPT,), jnp.float32),
           pltpu.VMEM_SHARED((NP,), jnp.float32),
           pltpu.VMEM_SHARED((NP, DH), jnp.float32)]
        + [pltpu.SemaphoreType.DMA] * (2 * NB + 1)
    ),
    compiler_params=pltpu.CompilerParams(use_tc_tiling_on_sc=False,
                                         needs_layout_passes=False),
)


def kernel(x2, edge_index, emb_weight):
    del x2
    row16 = edge_index[0].astype(jnp.int32).reshape(NTILE, NCHT, CH)
    col16 = edge_index[1].astype(jnp.int32).reshape(NTILE, NCHT, CH)
    emb_pad = jnp.zeros((NP, D), jnp.float32).at[:N].set(emb_weight)
    emb2 = jnp.stack([emb_pad[:, :DH], emb_pad[:, DH:]])

    osum, _, _ = _mega_kernel(emb2, row16, col16)
    return jnp.concatenate([osum[0, :N], osum[1, :N]], axis=1)


# speedup vs baseline: 1.3284x; 1.0250x over previous
import jax
import jax.numpy as jnp
from jax import lax
from jax.experimental import pallas as pl
from jax.experimental.pallas import tpu as pltpu
from jax.experimental.pallas import tpu_sc as plsc

N = 10000
E = 320000
D = 128
LAYERS = 3

NP = 10240
NSC = 2
NTILE = 16
CH = 40
CHP = 48
NCHT = E // CH // NTILE
DH = D // NSC
RPT = NP // NTILE
ZB = 64
NBLK = RPT // ZB

NB = 10
GK = 8

_mesh = plsc.VectorSubcoreMesh(core_axis_name="c", subcore_axis_name="s")


def _rsqrt16(x):
    i = plsc.bitcast(x, jnp.int32)
    y = plsc.bitcast(jnp.int32(0x5F3759DF) - (i >> 1), jnp.float32)
    for _ in range(3):
        y = y * (1.5 - 0.5 * x * y * y)
    return jnp.where(x > 0.0, y, 0.0)


def _mega_body(emb2_hbm, row16_hbm, col16_hbm,
               osum_hbm, zp_hbm, zq_hbm,
               ridx, cidx, *bufs):
    rows = list(bufs[:NB])
    zbuf = bufs[NB]
    zerob = bufs[NB + 1]
    obuf = bufs[NB + 2]
    ones_v = bufs[NB + 3]
    dis_v = bufs[NB + 4]
    deg_sh = bufs[NB + 5]
    acc_sh = bufs[NB + 6]
    hsem = bufs[NB + 7]
    gsem = list(bufs[NB + 8:2 * NB + 8])
    ssem = list(bufs[2 * NB + 8:])
    c = lax.axis_index("c")
    s = lax.axis_index("s")
    rr0 = s * RPT

    pltpu.sync_copy(row16_hbm.at[s], ridx)
    pltpu.sync_copy(col16_hbm.at[s], cidx)

    for j in range(CHP // 16):
        ones_v[pl.ds(j * 16, 16)] = jnp.ones((16,), jnp.float32)

    def zb(i, carry):
        for j in range(DH // 16):
            zerob[i, pl.ds(j * 16, 16)] = jnp.zeros((16,), jnp.float32)
        return carry

    lax.fori_loop(0, ZB, zb, 0)

    def zd(i, carry):
        dis_v[pl.ds(i * 16, 16)] = jnp.zeros((16,), jnp.float32)
        return carry

    lax.fori_loop(0, RPT // 16, zd, 0)
    pltpu.sync_copy(dis_v, deg_sh.at[pl.ds(rr0, RPT)])

    def zc(i, carry):
        pltpu.sync_copy(zerob, acc_sh.at[pl.ds(rr0 + i * ZB, ZB)])
        return carry

    lax.fori_loop(0, NBLK, zc, 0)
    plsc.subcore_barrier()

    HK = 25

    def hbatch(b, carry):
        def fire(i, carry2):
            pltpu.async_copy(ones_v.at[pl.ds(0, CH)],
                             deg_sh.at[cidx.at[b * HK + i]], hsem,
                             add=True)
            return carry2

        lax.fori_loop(0, HK, fire, 0)

        def drain(i, carry2):
            pltpu.make_async_copy(ones_v.at[pl.ds(0, CH)],
                                  deg_sh.at[cidx.at[0]], hsem).wait()
            return carry2

        lax.fori_loop(0, HK, drain, 0)
        return carry

    lax.fori_loop(0, NCHT // HK, hbatch, 0)
    plsc.subcore_barrier()

    pltpu.sync_copy(deg_sh.at[pl.ds(rr0, RPT)], dis_v)

    def newt(k, carry):
        x = dis_v[pl.ds(k * 16, 16)]
        dis_v[pl.ds(k * 16, 16)] = _rsqrt16(x)
        return carry

    lax.fori_loop(0, RPT // 16, newt, 0)

    def z0blk(i, carry):
        rr = rr0 + i * ZB
        pltpu.sync_copy(emb2_hbm.at[c, pl.ds(rr, ZB)], zbuf)

        def rowgrp(g, carry2):
            dv = dis_v[pl.ds(i * ZB + g * 16, 16)]
            for k in range(16):
                r = g * 16 + k
                d = dv[k]
                for j in range(DH // 16):
                    a = zbuf[r, pl.ds(j * 16, 16)]
                    zbuf[r, pl.ds(j * 16, 16)] = d * a
            return carry2

        lax.fori_loop(0, ZB // 16, rowgrp, 0)
        pltpu.sync_copy(zbuf, zp_hbm.at[c, pl.ds(rr, ZB)])
        return carry

    lax.fori_loop(0, NBLK, z0blk, 0)
    plsc.subcore_barrier()

    def sweep(zsrc):
        def gather(i, b):
            pltpu.async_copy(zsrc.at[ridx.at[i]], rows[b], gsem[b])

        def gwait(b):
            pltpu.make_async_copy(zsrc.at[ridx.at[0]], rows[b],
                                  gsem[b]).wait()

        def scat(i, b):
            pltpu.async_copy(rows[b], acc_sh.at[cidx.at[i]], ssem[b],
                             add=True)

        def swait(b):
            pltpu.make_async_copy(rows[b], acc_sh.at[cidx.at[0]],
                                  ssem[b]).wait()

        for b in range(GK):
            gather(b, b)

        def step(go, first, last):
            g = go * NB
            for b in range(NB):
                i = g + b
                gwait(b)
                scat(i, b)
                bk = (b + GK) % NB
                if last and b >= NB - GK:
                    continue
                if not (first and b < NB - GK):
                    swait(bk)
                gather(i + GK, bk)

        step(0, True, False)

        def mid(go, carry):
            step(go, False, False)
            return carry

        lax.fori_loop(1, NCHT // NB - 1, mid, 0)
        step(NCHT // NB - 1, False, True)
        for b in range(NB):
            swait(b)

    for layer in range(LAYERS):
        zin, zot = (zp_hbm, zq_hbm) if layer % 2 == 0 else (zq_hbm, zp_hbm)
        sweep(zin.at[c])
        plsc.subcore_barrier()

        last_layer = layer == LAYERS - 1

        first_layer = layer == 0

        def wo(i, carry):
            rr = rr0 + i * ZB
            pltpu.sync_copy(acc_sh.at[pl.ds(rr, ZB)], zbuf)
            if not first_layer:
                pltpu.sync_copy(osum_hbm.at[c, pl.ds(rr, ZB)], obuf)

            def rowgrp(g, carry2):
                dv = dis_v[pl.ds(i * ZB + g * 16, 16)]
                for k in range(16):
                    r = g * 16 + k
                    d = dv[k]
                    d2 = d * d
                    d3 = d * (1.0 / LAYERS)
                    for j in range(DH // 16):
                        a = zbuf[r, pl.ds(j * 16, 16)]
                        if first_layer:
                            obuf[r, pl.ds(j * 16, 16)] = d3 * a
                        else:
                            o = obuf[r, pl.ds(j * 16, 16)]
                            obuf[r, pl.ds(j * 16, 16)] = o + d3 * a
                        if not last_layer:
                            zbuf[r, pl.ds(j * 16, 16)] = d2 * a
                return carry2

            lax.fori_loop(0, ZB // 16, rowgrp, 0)
            pltpu.sync_copy(obuf, osum_hbm.at[c, pl.ds(rr, ZB)])
            if not last_layer:
                pltpu.sync_copy(zbuf, zot.at[c, pl.ds(rr, ZB)])
                pltpu.sync_copy(zerob, acc_sh.at[pl.ds(rr, ZB)])
            return carry

        lax.fori_loop(0, NBLK, wo, 0)
        plsc.subcore_barrier()


_mega_kernel = pl.kernel(
    _mega_body,
    out_type=[jax.ShapeDtypeStruct((NSC, NP, DH), jnp.float32),
              jax.ShapeDtypeStruct((NSC, NP, DH), jnp.float32),
              jax.ShapeDtypeStruct((NSC, NP, DH), jnp.float32)],
    mesh=_mesh,
    scratch_types=(
        [pltpu.VMEM((NCHT, CH), jnp.int32)] * 2
        + [pltpu.VMEM((CH, DH), jnp.float32)] * NB
        + [pltpu.VMEM((ZB, DH), jnp.float32),
           pltpu.VMEM((ZB, DH), jnp.float32),
           pltpu.VMEM((ZB, DH), jnp.float32),
           pltpu.VMEM((CHP,), jnp.float32),
           pltpu.VMEM((RPT,), jnp.float32),
           pltpu.VMEM_SHARED((NP,), jnp.float32),
           pltpu.VMEM_SHARED((NP, DH), jnp.float32)]
        + [pltpu.SemaphoreType.DMA] * (2 * NB + 1)
    ),
    compiler_params=pltpu.CompilerParams(use_tc_tiling_on_sc=False,
                                         needs_layout_passes=False),
)


def kernel(x2, edge_index, emb_weight):
    del x2
    row16 = edge_index[0].astype(jnp.int32).reshape(NTILE, NCHT, CH)
    col16 = edge_index[1].astype(jnp.int32).reshape(NTILE, NCHT, CH)
    emb_pad = jnp.zeros((NP, D), jnp.float32).at[:N].set(emb_weight)
    emb2 = jnp.stack([emb_pad[:, :DH], emb_pad[:, DH:]])

    osum, _, _ = _mega_kernel(emb2, row16, col16)
    return jnp.concatenate([osum[0, :N], osum[1, :N]], axis=1)
